# baseline (speedup 1.0000x reference)
import functools

import jax
import jax.numpy as jnp
from jax import lax
from jax.experimental import pallas as pl
from jax.experimental.pallas import tpu as pltpu
from jax.experimental.pallas import tpu_sc as plsc

_N = 10000
_E = 320000
_D = 128
_G = 64
_NW = 32
_EPW = _E // _NW
_C = 192
_NFULL = _EPW // _C
_CT = _EPW - _NFULL * _C
_R = 2
_LOOP_END = (_NFULL // _R) * _R
_RPT = 632
_RPT_LAST = _N - 15 * _RPT


def _segsum_sc(h, src, dst, zeros):
    mesh = plsc.VectorSubcoreMesh(core_axis_name="c", subcore_axis_name="s")

    @functools.partial(
        pl.kernel,
        out_type=jax.ShapeDtypeStruct((2, _N, _D), jnp.float32),
        mesh=mesh,
        scratch_types=(
            [pltpu.VMEM((_C,), jnp.int32) for _ in range(_R)]
            + [pltpu.VMEM((_C,), jnp.int32) for _ in range(_R)]
            + [pltpu.VMEM((_C, _D), jnp.float32) for _ in range(_R)]
            + [pltpu.VMEM_SHARED((_N, _D), jnp.float32)]
            + [pltpu.SemaphoreType.DMA for _ in range(2 * _R)]
            + [pltpu.VMEM((_CT,), jnp.int32),
               pltpu.VMEM((_CT,), jnp.int32)]
        ),
    )
    def seg_kernel(h_hbm, src_hbm, dst_hbm, zero_hbm, out_hbm, *scratch):
        sidx = scratch[0:_R]
        didx = scratch[_R:2 * _R]
        rows = scratch[2 * _R:3 * _R]
        shared = scratch[3 * _R]
        gsem = scratch[3 * _R + 1:4 * _R + 1]
        ssem = scratch[4 * _R + 1:5 * _R + 1]
        sidx_t, didx_t = scratch[5 * _R + 1], scratch[5 * _R + 2]
        bufs = tuple((sidx[r], didx[r], rows[r], gsem[r], ssem[r])
                     for r in range(_R))
        cid = lax.axis_index("c")
        sid = lax.axis_index("s")
        w = sid * 2 + cid

        def for_my_rows(fn):
            @pl.when(sid < 15)
            def _():
                fn(pl.ds(sid * _RPT, _RPT))

            @pl.when(sid == 15)
            def _():
                fn(pl.ds(15 * _RPT, _RPT_LAST))

        def init_rows(rslice):
            @pl.when(cid == 0)
            def _():
                pltpu.sync_copy(h_hbm.at[rslice], shared.at[rslice])

            @pl.when(cid != 0)
            def _():
                pltpu.sync_copy(zero_hbm.at[rslice], shared.at[rslice])

        ebase = w * _EPW

        def idx_copy(i, b):
            base = ebase + i * _C
            pltpu.sync_copy(src_hbm.at[pl.ds(base, _C)], b[0])
            pltpu.sync_copy(dst_hbm.at[pl.ds(base, _C)], b[1])

        def gstart(b):
            pltpu.async_copy(h_hbm.at[b[0]], b[2], b[3])

        def gwait(b):
            pltpu.make_async_copy(h_hbm.at[b[0]], b[2], b[3]).wait()

        def sstart(b):
            pltpu.async_copy(b[2], shared.at[b[1]], b[4], add=True)

        def swait(b):
            pltpu.make_async_copy(b[2], shared.at[b[1]], b[4]).wait()

        for j in range(_R - 1):
            idx_copy(j, bufs[j])
            gstart(bufs[j])
        for_my_rows(init_rows)
        plsc.subcore_barrier()

        @pl.loop(0, _LOOP_END, step=_R)
        def _(i):
            for k in range(_R):
                j = i + k
                b = bufs[k]
                prev = bufs[(k - 1) % _R]
                gwait(b)
                sstart(b)

                def refill(j=j, prev=prev):
                    idx_copy(j + _R - 1, prev)
                    gstart(prev)

                if k == 0:
                    @pl.when(i >= 1)
                    def _():
                        swait(prev)

                    @pl.when(i + _R - 1 < _NFULL)
                    def _():
                        refill()
                else:
                    swait(prev)

                    @pl.when(j + _R - 1 < _NFULL)
                    def _():
                        refill()

        swait(bufs[(_NFULL - 1) % _R])

        tbase = ebase + _NFULL * _C
        rows_t = bufs[0][2].at[pl.ds(0, _CT)]
        pltpu.sync_copy(src_hbm.at[pl.ds(tbase, _CT)], sidx_t)
        pltpu.sync_copy(dst_hbm.at[pl.ds(tbase, _CT)], didx_t)
        pltpu.async_copy(h_hbm.at[sidx_t], rows_t, bufs[0][3]).wait()
        pltpu.sync_copy(rows_t, shared.at[didx_t], add=True)

        plsc.subcore_barrier()
        for_my_rows(
            lambda rslice: pltpu.sync_copy(shared.at[rslice],
                                           out_hbm.at[cid].at[rslice]))

    return seg_kernel(h, src, dst, zeros)


def _layer_body(p_ref, w_ref, b_ref, g_ref, be_ref, o_ref):
    a = p_ref[0] + p_ref[1]
    pre = lax.dot_general(
        a, w_ref[...], (((1,), (0,)), ((), ())),
        precision=lax.Precision.HIGHEST,
        preferred_element_type=jnp.float32,
    ) + b_ref[...][None, :]
    mu = jnp.mean(pre, axis=0, keepdims=True)
    var = jnp.mean((pre - mu) ** 2, axis=0, keepdims=True)
    hn = (pre - mu) / jnp.sqrt(var + 1e-5) * g_ref[...][None, :] + be_ref[...][None, :]
    o_ref[...] = jnp.maximum(hn, 0.0)


def _tc_layer(parts, W, b, g, be):
    return pl.pallas_call(
        _layer_body,
        out_shape=jax.ShapeDtypeStruct((_N, _D), jnp.float32),
    )(parts, W, b, g, be)


def _head_body(h1_ref, bat1_ref, h2_ref, bat2_ref, wf_ref, bf_ref,
               wc1_ref, bc1_ref, wc2_ref, bc2_ref, o_ref):
    def pooled_emb(h_ref, bat_ref):
        bat = bat_ref[...]
        gids = lax.broadcasted_iota(jnp.int32, (_G, _N), 0)
        m = (bat[None, :] == gids).astype(jnp.float32)
        sums = lax.dot_general(
            m, h_ref[...], (((1,), (0,)), ((), ())),
            precision=lax.Precision.HIGHEST,
            preferred_element_type=jnp.float32,
        )
        cnt = jnp.sum(m, axis=1, keepdims=True)
        pooled = sums / jnp.maximum(cnt, 1.0)
        return lax.dot_general(
            pooled, wf_ref[...], (((1,), (0,)), ((), ())),
            precision=lax.Precision.HIGHEST,
            preferred_element_type=jnp.float32,
        ) + bf_ref[...][None, :]

    v1 = pooled_emb(h1_ref, bat1_ref)
    v2 = pooled_emb(h2_ref, bat2_ref)
    d = jnp.abs(v1 - v2)
    z = lax.dot_general(
        d, wc1_ref[...], (((1,), (0,)), ((), ())),
        precision=lax.Precision.HIGHEST,
        preferred_element_type=jnp.float32,
    ) + bc1_ref[...][None, :]
    z = jnp.maximum(z, 0.0)
    s = jnp.sum(z * wc2_ref[...][:, 0][None, :], axis=1, keepdims=True)
    s = s + bc2_ref[...][None, :]
    o_ref[...] = jax.nn.sigmoid(s)


def _head(h1, bat1, h2, bat2, Wf, bf, Wc1, bc1, Wc2, bc2):
    return pl.pallas_call(
        _head_body,
        out_shape=jax.ShapeDtypeStruct((_G, 1), jnp.float32),
    )(h1, bat1, h2, bat2, Wf, bf, Wc1, bc1, Wc2, bc2)


def kernel(x1, edge_index1, batch1, x2, edge_index2, batch2,
           W1, b1, g1, be1, W2, b2, g2, be2, W3, b3, g3, be3,
           Wf, bf, Wc1, bc1, Wc2, bc2):
    zeros = jnp.zeros((_N, _D), jnp.float32)
    layer_weights = ((W1, b1, g1, be1), (W2, b2, g2, be2), (W3, b3, g3, be3))

    def enc(x, ei):
        src, dst = ei[0], ei[1]
        h = x
        for (W, b, g, be) in layer_weights:
            parts = _segsum_sc(h, src, dst, zeros)
            h = _tc_layer(parts, W, b, g, be)
        return h

    h1 = enc(x1, edge_index1)
    h2 = enc(x2, edge_index2)
    return _head(h1, batch1, h2, batch2, Wf, bf, Wc1, bc1, Wc2, bc2)


# speedup vs baseline: 1.2516x; 1.2516x over previous
import functools

import jax
import jax.numpy as jnp
from jax import lax
from jax.experimental import pallas as pl
from jax.experimental.pallas import tpu as pltpu
from jax.experimental.pallas import tpu_sc as plsc

_N = 10000
_E = 320000
_D = 128
_G = 64
_NW = 32
_EPW = _E // _NW
_C = 96
_NFULL = _EPW // _C
_CT = _EPW - _NFULL * _C
_R = 4
_LOOP_END = (_NFULL // _R) * _R
_RPT = 632
_RPT_LAST = _N - 15 * _RPT


def _segsum_sc(h, src, dst, zeros):
    mesh = plsc.VectorSubcoreMesh(core_axis_name="c", subcore_axis_name="s")

    @functools.partial(
        pl.kernel,
        out_type=jax.ShapeDtypeStruct((2, _N, _D), jnp.float32),
        mesh=mesh,
        scratch_types=(
            [pltpu.VMEM((_C,), jnp.int32) for _ in range(_R)]
            + [pltpu.VMEM((_C,), jnp.int32) for _ in range(_R)]
            + [pltpu.VMEM((_C, _D), jnp.float32) for _ in range(_R)]
            + [pltpu.VMEM_SHARED((_N, _D), jnp.float32)]
            + [pltpu.SemaphoreType.DMA for _ in range(2 * _R)]
            + [pltpu.VMEM((_CT,), jnp.int32),
               pltpu.VMEM((_CT,), jnp.int32)]
        ),
    )
    def seg_kernel(h_hbm, src_hbm, dst_hbm, zero_hbm, out_hbm, *scratch):
        sidx = scratch[0:_R]
        didx = scratch[_R:2 * _R]
        rows = scratch[2 * _R:3 * _R]
        shared = scratch[3 * _R]
        gsem = scratch[3 * _R + 1:4 * _R + 1]
        ssem = scratch[4 * _R + 1:5 * _R + 1]
        sidx_t, didx_t = scratch[5 * _R + 1], scratch[5 * _R + 2]
        bufs = tuple((sidx[r], didx[r], rows[r], gsem[r], ssem[r])
                     for r in range(_R))
        cid = lax.axis_index("c")
        sid = lax.axis_index("s")
        w = sid * 2 + cid

        def for_my_rows(fn):
            @pl.when(sid < 15)
            def _():
                fn(pl.ds(sid * _RPT, _RPT))

            @pl.when(sid == 15)
            def _():
                fn(pl.ds(15 * _RPT, _RPT_LAST))

        def init_rows(rslice):
            @pl.when(cid == 0)
            def _():
                pltpu.sync_copy(h_hbm.at[rslice], shared.at[rslice])

            @pl.when(cid != 0)
            def _():
                pltpu.sync_copy(zero_hbm.at[rslice], shared.at[rslice])

        ebase = w * _EPW

        def idx_copy(i, b):
            base = ebase + i * _C
            pltpu.sync_copy(src_hbm.at[pl.ds(base, _C)], b[0])
            pltpu.sync_copy(dst_hbm.at[pl.ds(base, _C)], b[1])

        def gstart(b):
            pltpu.async_copy(h_hbm.at[b[0]], b[2], b[3])

        def gwait(b):
            pltpu.make_async_copy(h_hbm.at[b[0]], b[2], b[3]).wait()

        def sstart(b):
            pltpu.async_copy(b[2], shared.at[b[1]], b[4], add=True)

        def swait(b):
            pltpu.make_async_copy(b[2], shared.at[b[1]], b[4]).wait()

        for j in range(_R - 1):
            idx_copy(j, bufs[j])
            gstart(bufs[j])
        for_my_rows(init_rows)
        plsc.subcore_barrier()

        @pl.loop(0, _LOOP_END, step=_R)
        def _(i):
            for k in range(_R):
                j = i + k
                b = bufs[k]
                prev = bufs[(k - 1) % _R]
                gwait(b)
                sstart(b)

                def refill(j=j, prev=prev):
                    idx_copy(j + _R - 1, prev)
                    gstart(prev)

                if k == 0:
                    @pl.when(i >= 1)
                    def _():
                        swait(prev)

                    @pl.when(i + _R - 1 < _NFULL)
                    def _():
                        refill()
                else:
                    swait(prev)

                    @pl.when(j + _R - 1 < _NFULL)
                    def _():
                        refill()

        swait(bufs[(_NFULL - 1) % _R])

        tbase = ebase + _NFULL * _C
        rows_t = bufs[0][2].at[pl.ds(0, _CT)]
        pltpu.sync_copy(src_hbm.at[pl.ds(tbase, _CT)], sidx_t)
        pltpu.sync_copy(dst_hbm.at[pl.ds(tbase, _CT)], didx_t)
        pltpu.async_copy(h_hbm.at[sidx_t], rows_t, bufs[0][3]).wait()
        pltpu.sync_copy(rows_t, shared.at[didx_t], add=True)

        plsc.subcore_barrier()
        for_my_rows(
            lambda rslice: pltpu.sync_copy(shared.at[rslice],
                                           out_hbm.at[cid].at[rslice]))

    return seg_kernel(h, src, dst, zeros)


def _layer_body(p_ref, w_ref, b_ref, g_ref, be_ref, o_ref):
    a = p_ref[0] + p_ref[1]
    pre = lax.dot_general(
        a, w_ref[...], (((1,), (0,)), ((), ())),
        precision=lax.Precision.HIGHEST,
        preferred_element_type=jnp.float32,
    ) + b_ref[...][None, :]
    mu = jnp.mean(pre, axis=0, keepdims=True)
    var = jnp.mean((pre - mu) ** 2, axis=0, keepdims=True)
    hn = (pre - mu) / jnp.sqrt(var + 1e-5) * g_ref[...][None, :] + be_ref[...][None, :]
    o_ref[...] = jnp.maximum(hn, 0.0)


def _tc_layer(parts, W, b, g, be):
    return pl.pallas_call(
        _layer_body,
        out_shape=jax.ShapeDtypeStruct((_N, _D), jnp.float32),
    )(parts, W, b, g, be)


def _head_body(h1_ref, bat1_ref, h2_ref, bat2_ref, wf_ref, bf_ref,
               wc1_ref, bc1_ref, wc2_ref, bc2_ref, o_ref):
    def pooled_emb(h_ref, bat_ref):
        bat = bat_ref[...]
        gids = lax.broadcasted_iota(jnp.int32, (_G, _N), 0)
        m = (bat[None, :] == gids).astype(jnp.float32)
        sums = lax.dot_general(
            m, h_ref[...], (((1,), (0,)), ((), ())),
            precision=lax.Precision.HIGHEST,
            preferred_element_type=jnp.float32,
        )
        cnt = jnp.sum(m, axis=1, keepdims=True)
        pooled = sums / jnp.maximum(cnt, 1.0)
        return lax.dot_general(
            pooled, wf_ref[...], (((1,), (0,)), ((), ())),
            precision=lax.Precision.HIGHEST,
            preferred_element_type=jnp.float32,
        ) + bf_ref[...][None, :]

    v1 = pooled_emb(h1_ref, bat1_ref)
    v2 = pooled_emb(h2_ref, bat2_ref)
    d = jnp.abs(v1 - v2)
    z = lax.dot_general(
        d, wc1_ref[...], (((1,), (0,)), ((), ())),
        precision=lax.Precision.HIGHEST,
        preferred_element_type=jnp.float32,
    ) + bc1_ref[...][None, :]
    z = jnp.maximum(z, 0.0)
    s = jnp.sum(z * wc2_ref[...][:, 0][None, :], axis=1, keepdims=True)
    s = s + bc2_ref[...][None, :]
    o_ref[...] = jax.nn.sigmoid(s)


def _head(h1, bat1, h2, bat2, Wf, bf, Wc1, bc1, Wc2, bc2):
    return pl.pallas_call(
        _head_body,
        out_shape=jax.ShapeDtypeStruct((_G, 1), jnp.float32),
    )(h1, bat1, h2, bat2, Wf, bf, Wc1, bc1, Wc2, bc2)


def kernel(x1, edge_index1, batch1, x2, edge_index2, batch2,
           W1, b1, g1, be1, W2, b2, g2, be2, W3, b3, g3, be3,
           Wf, bf, Wc1, bc1, Wc2, bc2):
    zeros = jnp.zeros((_N, _D), jnp.float32)
    layer_weights = ((W1, b1, g1, be1), (W2, b2, g2, be2), (W3, b3, g3, be3))

    def enc(x, ei):
        src, dst = ei[0], ei[1]
        h = x
        for (W, b, g, be) in layer_weights:
            parts = _segsum_sc(h, src, dst, zeros)
            h = _tc_layer(parts, W, b, g, be)
        return h

    h1 = enc(x1, edge_index1)
    h2 = enc(x2, edge_index2)
    return _head(h1, batch1, h2, batch2, Wf, bf, Wc1, bc1, Wc2, bc2)


# speedup vs baseline: 1.3355x; 1.0671x over previous
import functools

import jax
import jax.numpy as jnp
from jax import lax
from jax.experimental import pallas as pl
from jax.experimental.pallas import tpu as pltpu
from jax.experimental.pallas import tpu_sc as plsc

_N = 10000
_E = 320000
_D = 128
_G = 64
_NWPS = 16
_EPW = _E // _NWPS
_C = 128
_NFULL = _EPW // _C
_CT = _EPW - _NFULL * _C
_R = 3
_LOOP_END = (_NFULL // _R) * _R
_RPT = 632
_RPT_LAST = _N - 15 * _RPT


def _segsum_sc(h1, src1, dst1, h2, src2, dst2):
    mesh = plsc.VectorSubcoreMesh(core_axis_name="c", subcore_axis_name="s")

    @functools.partial(
        pl.kernel,
        out_type=jax.ShapeDtypeStruct((2, _N, _D), jnp.float32),
        mesh=mesh,
        scratch_types=(
            [pltpu.VMEM((_C,), jnp.int32) for _ in range(_R)]
            + [pltpu.VMEM((_C,), jnp.int32) for _ in range(_R)]
            + [pltpu.VMEM((_C, _D), jnp.float32) for _ in range(_R)]
            + [pltpu.VMEM_SHARED((_N, _D), jnp.float32)]
            + [pltpu.SemaphoreType.DMA for _ in range(2 * _R)]
            + [pltpu.VMEM((_CT,), jnp.int32),
               pltpu.VMEM((_CT,), jnp.int32)]
        ),
    )
    def seg_kernel(h1_hbm, src1_hbm, dst1_hbm, h2_hbm, src2_hbm, dst2_hbm,
                   out_hbm, *scratch):
        sidx = scratch[0:_R]
        didx = scratch[_R:2 * _R]
        rows = scratch[2 * _R:3 * _R]
        shared = scratch[3 * _R]
        gsem = scratch[3 * _R + 1:4 * _R + 1]
        ssem = scratch[4 * _R + 1:5 * _R + 1]
        sidx_t, didx_t = scratch[5 * _R + 1], scratch[5 * _R + 2]
        bufs = tuple((sidx[r], didx[r], rows[r], gsem[r], ssem[r])
                     for r in range(_R))
        cid = lax.axis_index("c")
        sid = lax.axis_index("s")

        def for_my_rows(fn):
            @pl.when(sid < 15)
            def _():
                fn(pl.ds(sid * _RPT, _RPT))

            @pl.when(sid == 15)
            def _():
                fn(pl.ds(15 * _RPT, _RPT_LAST))

        def run_side(h_hbm, src_hbm, dst_hbm, out_side):
            ebase = sid * _EPW

            def idx_copy(i, b):
                base = ebase + i * _C
                pltpu.sync_copy(src_hbm.at[pl.ds(base, _C)], b[0])
                pltpu.sync_copy(dst_hbm.at[pl.ds(base, _C)], b[1])

            def gstart(b):
                pltpu.async_copy(h_hbm.at[b[0]], b[2], b[3])

            def gwait(b):
                pltpu.make_async_copy(h_hbm.at[b[0]], b[2], b[3]).wait()

            def sstart(b):
                pltpu.async_copy(b[2], shared.at[b[1]], b[4], add=True)

            def swait(b):
                pltpu.make_async_copy(b[2], shared.at[b[1]], b[4]).wait()

            for j in range(_R - 1):
                idx_copy(j, bufs[j])
                gstart(bufs[j])
            for_my_rows(
                lambda rs: pltpu.sync_copy(h_hbm.at[rs], shared.at[rs]))
            plsc.subcore_barrier()

            @pl.loop(0, _LOOP_END, step=_R)
            def _(i):
                for k in range(_R):
                    j = i + k
                    b = bufs[k]
                    prev = bufs[(k - 1) % _R]
                    gwait(b)
                    sstart(b)

                    def refill(j=j, prev=prev):
                        idx_copy(j + _R - 1, prev)
                        gstart(prev)

                    if k == 0:
                        @pl.when(i >= 1)
                        def _():
                            swait(prev)

                        @pl.when(i + _R - 1 < _NFULL)
                        def _():
                            refill()
                    else:
                        swait(prev)

                        @pl.when(j + _R - 1 < _NFULL)
                        def _():
                            refill()

            swait(bufs[(_NFULL - 1) % _R])

            tbase = ebase + _NFULL * _C
            rows_t = bufs[0][2].at[pl.ds(0, _CT)]
            pltpu.sync_copy(src_hbm.at[pl.ds(tbase, _CT)], sidx_t)
            pltpu.sync_copy(dst_hbm.at[pl.ds(tbase, _CT)], didx_t)
            pltpu.async_copy(h_hbm.at[sidx_t], rows_t, bufs[0][3]).wait()
            pltpu.sync_copy(rows_t, shared.at[didx_t], add=True)

            plsc.subcore_barrier()
            for_my_rows(
                lambda rs: pltpu.sync_copy(shared.at[rs], out_side.at[rs]))

        @pl.when(cid == 0)
        def _():
            run_side(h1_hbm, src1_hbm, dst1_hbm, out_hbm.at[0])

        @pl.when(cid != 0)
        def _():
            run_side(h2_hbm, src2_hbm, dst2_hbm, out_hbm.at[1])

    return seg_kernel(h1, src1, dst1, h2, src2, dst2)


def _layer_body(agg_ref, w_ref, b_ref, g_ref, be_ref, o1_ref, o2_ref):
    for s, o_ref in ((0, o1_ref), (1, o2_ref)):
        pre = lax.dot_general(
            agg_ref[s], w_ref[...], (((1,), (0,)), ((), ())),
            precision=lax.Precision.HIGHEST,
            preferred_element_type=jnp.float32,
        ) + b_ref[...][None, :]
        mu = jnp.mean(pre, axis=0, keepdims=True)
        var = jnp.mean((pre - mu) ** 2, axis=0, keepdims=True)
        hn = (pre - mu) / jnp.sqrt(var + 1e-5) * g_ref[...][None, :] \
            + be_ref[...][None, :]
        o_ref[...] = jnp.maximum(hn, 0.0)


def _tc_layer(agg, W, b, g, be):
    return pl.pallas_call(
        _layer_body,
        out_shape=(jax.ShapeDtypeStruct((_N, _D), jnp.float32),
                   jax.ShapeDtypeStruct((_N, _D), jnp.float32)),
    )(agg, W, b, g, be)


def _head_body(h1_ref, bat1_ref, h2_ref, bat2_ref, wf_ref, bf_ref,
               wc1_ref, bc1_ref, wc2_ref, bc2_ref, o_ref):
    def pooled_emb(h_ref, bat_ref):
        bat = bat_ref[...]
        gids = lax.broadcasted_iota(jnp.int32, (_G, _N), 0)
        m = (bat[None, :] == gids).astype(jnp.float32)
        sums = lax.dot_general(
            m, h_ref[...], (((1,), (0,)), ((), ())),
            precision=lax.Precision.HIGHEST,
            preferred_element_type=jnp.float32,
        )
        cnt = jnp.sum(m, axis=1, keepdims=True)
        pooled = sums / jnp.maximum(cnt, 1.0)
        return lax.dot_general(
            pooled, wf_ref[...], (((1,), (0,)), ((), ())),
            precision=lax.Precision.HIGHEST,
            preferred_element_type=jnp.float32,
        ) + bf_ref[...][None, :]

    v1 = pooled_emb(h1_ref, bat1_ref)
    v2 = pooled_emb(h2_ref, bat2_ref)
    d = jnp.abs(v1 - v2)
    z = lax.dot_general(
        d, wc1_ref[...], (((1,), (0,)), ((), ())),
        precision=lax.Precision.HIGHEST,
        preferred_element_type=jnp.float32,
    ) + bc1_ref[...][None, :]
    z = jnp.maximum(z, 0.0)
    s = jnp.sum(z * wc2_ref[...][:, 0][None, :], axis=1, keepdims=True)
    s = s + bc2_ref[...][None, :]
    o_ref[...] = jax.nn.sigmoid(s)


def _head(h1, bat1, h2, bat2, Wf, bf, Wc1, bc1, Wc2, bc2):
    return pl.pallas_call(
        _head_body,
        out_shape=jax.ShapeDtypeStruct((_G, 1), jnp.float32),
    )(h1, bat1, h2, bat2, Wf, bf, Wc1, bc1, Wc2, bc2)


def kernel(x1, edge_index1, batch1, x2, edge_index2, batch2,
           W1, b1, g1, be1, W2, b2, g2, be2, W3, b3, g3, be3,
           Wf, bf, Wc1, bc1, Wc2, bc2):
    src1, dst1 = edge_index1[0], edge_index1[1]
    src2, dst2 = edge_index2[0], edge_index2[1]
    h1, h2 = x1, x2
    for (W, b, g, be) in ((W1, b1, g1, be1), (W2, b2, g2, be2),
                          (W3, b3, g3, be3)):
        agg = _segsum_sc(h1, src1, dst1, h2, src2, dst2)
        h1, h2 = _tc_layer(agg, W, b, g, be)
    return _head(h1, batch1, h2, batch2, Wf, bf, Wc1, bc1, Wc2, bc2)


# speedup vs baseline: 1.5605x; 1.1684x over previous
import functools

import jax
import jax.numpy as jnp
from jax import lax
from jax.experimental import pallas as pl
from jax.experimental.pallas import tpu as pltpu
from jax.experimental.pallas import tpu_sc as plsc

_N = 10000
_E = 320000
_D = 128
_G = 64
_NWPS = 16
_C = 128
_NCH = _E // _C
_CPW = _NCH // _NWPS
_XTRA = _NCH - _CPW * _NWPS
_R = 3
_RI = 6
_RPT = 632
_RPT_LAST = _N - 15 * _RPT


def _segsum_sc(h1, ei1, h2, ei2):
    mesh = plsc.VectorSubcoreMesh(core_axis_name="c", subcore_axis_name="s")

    @functools.partial(
        pl.kernel,
        out_type=jax.ShapeDtypeStruct((2, _N, _D), jnp.float32),
        mesh=mesh,
        scratch_types=(
            [pltpu.VMEM((2, _C), jnp.int32) for _ in range(_RI)]
            + [pltpu.VMEM((_C, _D), jnp.float32) for _ in range(_R)]
            + [pltpu.VMEM_SHARED((_N, _D), jnp.float32)]
            + [pltpu.SemaphoreType.DMA for _ in range(_RI + 2 * _R)]
        ),
    )
    def seg_kernel(h1_hbm, ei1_hbm, h2_hbm, ei2_hbm, out_hbm, *scratch):
        eidx = scratch[0:_RI]
        rows = scratch[_RI:_RI + _R]
        shared = scratch[_RI + _R]
        sems = scratch[_RI + _R + 1:]
        isem = sems[0:_RI]
        gsem = sems[_RI:_RI + _R]
        ssem = sems[_RI + _R:_RI + 2 * _R]
        cid = lax.axis_index("c")
        sid = lax.axis_index("s")

        def for_my_rows(fn):
            @pl.when(sid < 15)
            def _():
                fn(pl.ds(sid * _RPT, _RPT))

            @pl.when(sid == 15)
            def _():
                fn(pl.ds(15 * _RPT, _RPT_LAST))

        def run_side(h_hbm, ei_hbm, out_side):
            cbase = sid * _CPW

            def idx_start(j, m):
                pltpu.async_copy(ei_hbm.at[cbase + j], eidx[m], isem[m])

            def idx_wait(j, m):
                pltpu.make_async_copy(
                    ei_hbm.at[cbase + j], eidx[m], isem[m]).wait()

            def gstart(m, r):
                pltpu.async_copy(h_hbm.at[eidx[m].at[0]], rows[r], gsem[r])

            def gwait(m, r):
                pltpu.make_async_copy(
                    h_hbm.at[eidx[m].at[0]], rows[r], gsem[r]).wait()

            def sstart(m, r):
                pltpu.async_copy(rows[r], shared.at[eidx[m].at[1]], ssem[r],
                                 add=True)

            def swait(m, r):
                pltpu.make_async_copy(
                    rows[r], shared.at[eidx[m].at[1]], ssem[r]).wait()

            for j in range(_RI - 1):
                idx_start(j, j)
            for j in range(_R - 1):
                idx_wait(j, j)
                gstart(j, j)
            for_my_rows(
                lambda rs: pltpu.sync_copy(h_hbm.at[rs], shared.at[rs]))
            plsc.subcore_barrier()

            def slot(j, k):
                m, r = k % _RI, k % _R
                gwait(m, r)
                sstart(m, r)

                first = (k == 0)

                def drain():
                    swait((k - 1) % _RI, (k - 1) % _R)

                def refill():
                    idx_start(j + _RI - 1, (k - 1) % _RI)

                def next_gather():
                    idx_wait(j + _R - 1, (k + _R - 1) % _RI)
                    gstart((k + _R - 1) % _RI, (k + _R - 1) % _R)

                if first:
                    @pl.when(j >= 1)
                    def _():
                        drain()
                else:
                    drain()

                @pl.when(j + _RI - 1 < _CPW)
                def _():
                    refill()

                @pl.when(j + _R - 1 < _CPW)
                def _():
                    next_gather()

            @pl.loop(0, _CPW, step=_RI)
            def _(i):
                for k in range(_RI):
                    slot(i + k, k)

            swait((_CPW - 1) % _RI, (_CPW - 1) % _R)

            @pl.when(sid < _XTRA)
            def _():
                xj = _NWPS * _CPW + sid
                pltpu.async_copy(ei_hbm.at[xj], eidx[0], isem[0])
                pltpu.make_async_copy(ei_hbm.at[xj], eidx[0], isem[0]).wait()
                pltpu.async_copy(h_hbm.at[eidx[0].at[0]], rows[0], gsem[0])
                pltpu.make_async_copy(
                    h_hbm.at[eidx[0].at[0]], rows[0], gsem[0]).wait()
                pltpu.sync_copy(rows[0], shared.at[eidx[0].at[1]], add=True)

            plsc.subcore_barrier()
            for_my_rows(
                lambda rs: pltpu.sync_copy(shared.at[rs], out_side.at[rs]))

        @pl.when(cid == 0)
        def _():
            run_side(h1_hbm, ei1_hbm, out_hbm.at[0])

        @pl.when(cid != 0)
        def _():
            run_side(h2_hbm, ei2_hbm, out_hbm.at[1])

    return seg_kernel(h1, ei1, h2, ei2)


def _layer_body(agg_ref, w_ref, b_ref, g_ref, be_ref, o1_ref, o2_ref):
    for s, o_ref in ((0, o1_ref), (1, o2_ref)):
        pre = lax.dot_general(
            agg_ref[s], w_ref[...], (((1,), (0,)), ((), ())),
            precision=lax.Precision.HIGHEST,
            preferred_element_type=jnp.float32,
        ) + b_ref[...][None, :]
        mu = jnp.mean(pre, axis=0, keepdims=True)
        var = jnp.mean((pre - mu) ** 2, axis=0, keepdims=True)
        hn = (pre - mu) / jnp.sqrt(var + 1e-5) * g_ref[...][None, :] \
            + be_ref[...][None, :]
        o_ref[...] = jnp.maximum(hn, 0.0)


def _tc_layer(agg, W, b, g, be):
    return pl.pallas_call(
        _layer_body,
        out_shape=(jax.ShapeDtypeStruct((_N, _D), jnp.float32),
                   jax.ShapeDtypeStruct((_N, _D), jnp.float32)),
    )(agg, W, b, g, be)


def _head_body(h1_ref, bat1_ref, h2_ref, bat2_ref, wf_ref, bf_ref,
               wc1_ref, bc1_ref, wc2_ref, bc2_ref, o_ref):
    def pooled_emb(h_ref, bat_ref):
        bat = bat_ref[...]
        gids = lax.broadcasted_iota(jnp.int32, (_G, _N), 0)
        m = (bat[None, :] == gids).astype(jnp.float32)
        sums = lax.dot_general(
            m, h_ref[...], (((1,), (0,)), ((), ())),
            precision=lax.Precision.HIGHEST,
            preferred_element_type=jnp.float32,
        )
        cnt = jnp.sum(m, axis=1, keepdims=True)
        pooled = sums / jnp.maximum(cnt, 1.0)
        return lax.dot_general(
            pooled, wf_ref[...], (((1,), (0,)), ((), ())),
            precision=lax.Precision.HIGHEST,
            preferred_element_type=jnp.float32,
        ) + bf_ref[...][None, :]

    v1 = pooled_emb(h1_ref, bat1_ref)
    v2 = pooled_emb(h2_ref, bat2_ref)
    d = jnp.abs(v1 - v2)
    z = lax.dot_general(
        d, wc1_ref[...], (((1,), (0,)), ((), ())),
        precision=lax.Precision.HIGHEST,
        preferred_element_type=jnp.float32,
    ) + bc1_ref[...][None, :]
    z = jnp.maximum(z, 0.0)
    s = jnp.sum(z * wc2_ref[...][:, 0][None, :], axis=1, keepdims=True)
    s = s + bc2_ref[...][None, :]
    o_ref[...] = jax.nn.sigmoid(s)


def _head(h1, bat1, h2, bat2, Wf, bf, Wc1, bc1, Wc2, bc2):
    return pl.pallas_call(
        _head_body,
        out_shape=jax.ShapeDtypeStruct((_G, 1), jnp.float32),
    )(h1, bat1, h2, bat2, Wf, bf, Wc1, bc1, Wc2, bc2)


def kernel(x1, edge_index1, batch1, x2, edge_index2, batch2,
           W1, b1, g1, be1, W2, b2, g2, be2, W3, b3, g3, be3,
           Wf, bf, Wc1, bc1, Wc2, bc2):
    ei1 = edge_index1.reshape(2, _NCH, _C).transpose(1, 0, 2)
    ei2 = edge_index2.reshape(2, _NCH, _C).transpose(1, 0, 2)
    h1, h2 = x1, x2
    for (W, b, g, be) in ((W1, b1, g1, be1), (W2, b2, g2, be2),
                          (W3, b3, g3, be3)):
        agg = _segsum_sc(h1, ei1, h2, ei2)
        h1, h2 = _tc_layer(agg, W, b, g, be)
    return _head(h1, batch1, h2, batch2, Wf, bf, Wc1, bc1, Wc2, bc2)
